# baseline (speedup 1.0000x reference)
import functools

import jax
import jax.numpy as jnp
from jax import lax
from jax.experimental import pallas as pl
from jax.experimental.pallas import tpu as pltpu
from jax.experimental.pallas import tpu_sc as plsc

N = 10000
D_IN = 128
D_OUT = 128
D_HALF = D_OUT // 2
NSC = 2
NSUB = 16
CHUNK = 128
ROWS_PER_SUB = N // NSUB
RELU_BLK = 125
NBUF = 3
NMETA = 6


def _matmul_body(x_ref, w_ref, o_ref):
    o_ref[0] = jnp.dot(
        x_ref[...], w_ref[0], preferred_element_type=jnp.float32
    ).astype(jnp.bfloat16)


def _matmul_split(x, w_split, row_blk):
    n = x.shape[0]
    grid = (NSC, n // row_blk)
    return pl.pallas_call(
        _matmul_body,
        grid=grid,
        in_specs=[
            pl.BlockSpec((row_blk, D_IN), lambda c, i: (i, 0)),
            pl.BlockSpec((1, D_IN, D_HALF), lambda c, i: (c, 0, 0)),
        ],
        out_specs=pl.BlockSpec((1, row_blk, D_HALF), lambda c, i: (c, i, 0)),
        out_shape=jax.ShapeDtypeStruct((NSC, n, D_HALF), jnp.bfloat16),
    )(x, w_split)


def _make_sc_kernel(n_chunks):
    assert n_chunks % NMETA == 0
    mesh = plsc.VectorSubcoreMesh(core_axis_name="c", subcore_axis_name="s")

    @functools.partial(
        pl.kernel,
        mesh=mesh,
        out_type=jax.ShapeDtypeStruct((N, D_OUT), jnp.float32),
        compiler_params=pltpu.CompilerParams(
            use_tc_tiling_on_sc=False, needs_layout_passes=False),
        scratch_types=[
            pltpu.VMEM((NMETA, 3, CHUNK), jnp.int32),
            pltpu.VMEM((NBUF, CHUNK, D_HALF // 2), jnp.int32),
            pltpu.VMEM((NBUF, CHUNK, D_HALF), jnp.float32),
            pltpu.VMEM_SHARED((N, D_HALF // 2), jnp.int32),
            pltpu.VMEM_SHARED((N, D_HALF), jnp.float32),
            pltpu.SemaphoreType.DMA((NMETA,)),
            pltpu.SemaphoreType.DMA((NBUF,)),
            pltpu.SemaphoreType.DMA((NBUF,)),
        ],
    )
    def spmm(h_hbm, meta_hbm, out_hbm,
             meta_v, rows16_v, rows_v, h_spm, acc, msem, gsem, ssem):
        c = lax.axis_index("c")
        s = lax.axis_index("s")

        h_base = c * N + s * ROWS_PER_SUB
        pltpu.sync_copy(h_hbm.at[pl.ds(h_base, ROWS_PER_SUB)],
                        h_spm.at[pl.ds(s * ROWS_PER_SUB, ROWS_PER_SUB)])

        @plsc.parallel_loop(0, CHUNK, unroll=4)
        def _(i):
            for k in range(D_HALF // 16):
                rows_v[0, i, pl.ds(16 * k, 16)] = jnp.zeros((16,), jnp.float32)

        for b in range(ROWS_PER_SUB // RELU_BLK):
            pltpu.sync_copy(
                rows_v.at[0, pl.ds(0, RELU_BLK)],
                acc.at[pl.ds(s * ROWS_PER_SUB + b * RELU_BLK, RELU_BLK)],
            )
        plsc.subcore_barrier()

        def start_meta(j, mj):
            pltpu.async_copy(meta_hbm.at[s, j], meta_v.at[mj], msem.at[mj])

        def start_gather(b, mj):
            pltpu.async_copy(h_spm.at[meta_v.at[mj, 0]], rows16_v.at[b],
                             gsem.at[b])

        for mj in range(NMETA):
            start_meta(mj, mj)
        for b in range(NBUF):
            pltpu.make_async_copy(meta_hbm.at[s, b], meta_v.at[b],
                                  msem.at[b]).wait()
            start_gather(b, b)

        def process(j, b, mj):
            mp = (mj + NBUF) % NMETA

            @pl.when(j >= NBUF)
            def _():
                pltpu.make_async_copy(rows_v.at[b], acc.at[meta_v.at[mp, 1]],
                                      ssem.at[b]).wait()
                @pl.when(j + NBUF < n_chunks)
                def _():
                    start_meta(j + NBUF, mp)

            pltpu.make_async_copy(h_spm.at[meta_v.at[mj, 0]], rows16_v.at[b],
                                  gsem.at[b]).wait()

            @plsc.parallel_loop(0, CHUNK // 16, unroll=2)
            def _(m):
                v16 = plsc.bitcast(meta_v[mj, 2, pl.ds(m * 16, 16)], jnp.float32)
                for r2 in range(16):
                    bc = jnp.broadcast_to(v16[r2], (16,))
                    r = m * 16 + r2
                    for t in range(D_HALF // 32):
                        xi = rows16_v[b, r, pl.ds(16 * t, 16)]
                        lo = plsc.bitcast(lax.shift_left(xi, 16), jnp.float32)
                        hi = plsc.bitcast(xi & jnp.int32(-65536), jnp.float32)
                        rows_v[b, r, pl.ds(32 * t, 16)] = lo * bc
                        rows_v[b, r, pl.ds(32 * t + 16, 16)] = hi * bc

            pltpu.async_copy(rows_v.at[b], acc.at[meta_v.at[mj, 1]],
                             ssem.at[b], add=True)

        def ring_body(g, _):
            for u in range(NMETA):
                j = g * NMETA + u
                b = u % NBUF
                process(j, b, u)
                br = (b + 2) % NBUF
                mg = (u + 2) % NMETA

                @pl.when(jnp.logical_and(j >= 1, j + 2 < n_chunks))
                def _():
                    pltpu.make_async_copy(meta_hbm.at[s, j + 2],
                                          meta_v.at[mg], msem.at[mg]).wait()
                    start_gather(br, mg)
            return ()

        lax.fori_loop(0, n_chunks // NMETA, ring_body, ())

        for b in range(NBUF):
            mj = (n_chunks - NBUF + b) % NMETA
            pltpu.make_async_copy(rows_v.at[b], acc.at[meta_v.at[mj, 1]],
                                  ssem.at[b]).wait()
        plsc.subcore_barrier()

        for b in range(ROWS_PER_SUB // RELU_BLK):
            row0 = s * ROWS_PER_SUB + b * RELU_BLK
            buf = b % NBUF
            pltpu.sync_copy(acc.at[pl.ds(row0, RELU_BLK)],
                            rows_v.at[buf, pl.ds(0, RELU_BLK)])

            @plsc.parallel_loop(0, RELU_BLK, unroll=4)
            def _(r):
                for k in range(D_HALF // 16):
                    sl = pl.ds(16 * k, 16)
                    rows_v[buf, r, sl] = jnp.maximum(rows_v[buf, r, sl], 0.0)

            pltpu.sync_copy(rows_v.at[buf, pl.ds(0, RELU_BLK)],
                            out_hbm.at[pl.ds(row0, RELU_BLK),
                                       pl.ds(c * D_HALF, D_HALF)])

    return spmm


def kernel(x, edge_index, adj_values, W):
    e = edge_index.shape[1]
    n_chunks = -(-e // (NSUB * CHUNK))
    n_chunks = -(-n_chunks // NMETA) * NMETA
    e_pad = NSUB * n_chunks * CHUNK
    pad = e_pad - e

    src = jnp.concatenate([edge_index[0], jnp.zeros((pad,), jnp.int32)])
    dst = jnp.concatenate([edge_index[1], jnp.zeros((pad,), jnp.int32)])
    val = jnp.concatenate([adj_values, jnp.zeros((pad,), jnp.float32)])
    vali = lax.bitcast_convert_type(val, jnp.int32)
    meta = jnp.stack(
        [src.reshape(NSUB, n_chunks, CHUNK),
         dst.reshape(NSUB, n_chunks, CHUNK),
         vali.reshape(NSUB, n_chunks, CHUNK)], axis=2)

    pos = jnp.arange(D_OUT)
    perm = (pos // 32) * 32 + (pos % 32) // 2 + 16 * (pos % 2)
    w_perm = W[:, perm]
    w_split = w_perm.reshape(D_IN, NSC, D_HALF).transpose(1, 0, 2)
    h_split = _matmul_split(x, w_split, row_blk=1000)
    h_flat = lax.bitcast_convert_type(
        h_split.reshape(NSC * N, D_HALF // 2, 2), jnp.int32)

    return _make_sc_kernel(n_chunks)(h_flat, meta)


# speedup vs baseline: 1.2439x; 1.2439x over previous
import functools

import jax
import jax.numpy as jnp
from jax import lax
from jax.experimental import pallas as pl
from jax.experimental.pallas import tpu as pltpu
from jax.experimental.pallas import tpu_sc as plsc

N = 10000
D_IN = 128
D_OUT = 128
D_HALF = D_OUT // 2
NSC = 2
NSUB = 16
CHUNK = 125
ROWS_PER_SUB = N // NSUB
RELU_BLK = 125
NBUF = 2
NMETA = 4


def _matmul_body(x_ref, w_ref, o_ref):
    o_ref[0] = jnp.dot(x_ref[...], w_ref[0], preferred_element_type=jnp.float32)


def _matmul_split(x, w_split, row_blk):
    n = x.shape[0]
    grid = (NSC, n // row_blk)
    return pl.pallas_call(
        _matmul_body,
        grid=grid,
        in_specs=[
            pl.BlockSpec((row_blk, D_IN), lambda c, i: (i, 0)),
            pl.BlockSpec((1, D_IN, D_HALF), lambda c, i: (c, 0, 0)),
        ],
        out_specs=pl.BlockSpec((1, row_blk, D_HALF), lambda c, i: (c, i, 0)),
        out_shape=jax.ShapeDtypeStruct((NSC, n, D_HALF), jnp.float32),
    )(x, w_split)


def _make_sc_kernel(n_chunks):
    assert n_chunks % NMETA == 0
    mesh = plsc.VectorSubcoreMesh(core_axis_name="c", subcore_axis_name="s")

    @functools.partial(
        pl.kernel,
        mesh=mesh,
        out_type=jax.ShapeDtypeStruct((N, D_OUT), jnp.float32),
        compiler_params=pltpu.CompilerParams(
            use_tc_tiling_on_sc=False, needs_layout_passes=False),
        scratch_types=[
            pltpu.VMEM((NMETA, CHUNK), jnp.int32),
            pltpu.VMEM((NMETA, CHUNK), jnp.int32),
            pltpu.VMEM((NMETA, CHUNK), jnp.float32),
            pltpu.VMEM((NBUF, CHUNK, D_HALF), jnp.float32),
            pltpu.VMEM_SHARED((N, D_HALF), jnp.float32),
            pltpu.VMEM_SHARED((N, D_HALF), jnp.float32),
            pltpu.SemaphoreType.DMA((NMETA,)),
            pltpu.SemaphoreType.DMA((NBUF,)),
            pltpu.SemaphoreType.DMA((NBUF,)),
        ],
    )
    def spmm(h_hbm, src_hbm, dst_hbm, val_hbm, out_hbm,
             src_v, dst_v, val_v, rows_v, h_spm, acc, msem, gsem, ssem):
        c = lax.axis_index("c")
        s = lax.axis_index("s")

        h_base = c * N + s * ROWS_PER_SUB
        pltpu.sync_copy(h_hbm.at[pl.ds(h_base, ROWS_PER_SUB)],
                        h_spm.at[pl.ds(s * ROWS_PER_SUB, ROWS_PER_SUB)])

        @plsc.parallel_loop(0, CHUNK, unroll=4)
        def _(i):
            for k in range(D_HALF // 16):
                rows_v[0, i, pl.ds(16 * k, 16)] = jnp.zeros((16,), jnp.float32)

        for b in range(ROWS_PER_SUB // RELU_BLK):
            pltpu.sync_copy(
                rows_v.at[0, pl.ds(0, RELU_BLK)],
                acc.at[pl.ds(s * ROWS_PER_SUB + b * RELU_BLK, RELU_BLK)],
            )
        plsc.subcore_barrier()

        def start_meta(j, mj):
            pltpu.async_copy(src_hbm.at[s, j], src_v.at[mj], msem.at[mj])
            pltpu.async_copy(dst_hbm.at[s, j], dst_v.at[mj], msem.at[mj])
            pltpu.async_copy(val_hbm.at[s, j], val_v.at[mj], msem.at[mj])

        def wait_meta(j, mj):
            pltpu.make_async_copy(src_hbm.at[s, j], src_v.at[mj],
                                  msem.at[mj]).wait()
            pltpu.make_async_copy(dst_hbm.at[s, j], dst_v.at[mj],
                                  msem.at[mj]).wait()
            pltpu.make_async_copy(val_hbm.at[s, j], val_v.at[mj],
                                  msem.at[mj]).wait()

        def start_gather(b, mj):
            pltpu.async_copy(h_spm.at[src_v.at[mj]], rows_v.at[b], gsem.at[b])

        for mj in range(NMETA):
            start_meta(mj, mj)
        wait_meta(0, 0)
        start_gather(0, 0)

        def process(j, b, mj):
            pltpu.make_async_copy(h_spm.at[src_v.at[mj]], rows_v.at[b],
                                  gsem.at[b]).wait()

            @plsc.parallel_loop(0, CHUNK // 16, unroll=2)
            def _(m):
                v16 = val_v[mj, pl.ds(m * 16, 16)]
                for r2 in range(16):
                    bc = jnp.broadcast_to(v16[r2], (16,))
                    r = m * 16 + r2
                    for k in range(D_HALF // 16):
                        sl = pl.ds(16 * k, 16)
                        rows_v[b, r, sl] = rows_v[b, r, sl] * bc

            v16t = val_v[mj, pl.ds(CHUNK - 16, 16)]
            for r2 in range(13):
                bc = jnp.broadcast_to(v16t[r2 + 3], (16,))
                r = 112 + r2
                for k in range(D_HALF // 16):
                    sl = pl.ds(16 * k, 16)
                    rows_v[b, r, sl] = rows_v[b, r, sl] * bc

            pltpu.async_copy(rows_v.at[b], acc.at[dst_v.at[mj]],
                             ssem.at[b], add=True)

        def ring_body(g, _):
            for u in range(NMETA):
                j = g * NMETA + u
                b = u % NBUF
                process(j, b, u)
                br = (u + 1) % NBUF
                mr = (u + 3) % NMETA
                mg = (u + 1) % NMETA

                @pl.when(j + 1 < n_chunks)
                def _():
                    @pl.when(j >= 1)
                    def _():
                        pltpu.make_async_copy(rows_v.at[br],
                                              acc.at[dst_v.at[mr]],
                                              ssem.at[br]).wait()

                        @pl.when(j + 3 < n_chunks)
                        def _():
                            start_meta(j + 3, mr)

                    wait_meta(j + 1, mg)
                    start_gather(br, mg)
            return ()

        lax.fori_loop(0, n_chunks // NMETA, ring_body, ())

        for b in range(NBUF):
            mj = (n_chunks - NBUF + b) % NMETA
            pltpu.make_async_copy(rows_v.at[b], acc.at[dst_v.at[mj]],
                                  ssem.at[b]).wait()
        plsc.subcore_barrier()

        for b in range(ROWS_PER_SUB // RELU_BLK):
            row0 = s * ROWS_PER_SUB + b * RELU_BLK
            buf = b % NBUF
            pltpu.sync_copy(acc.at[pl.ds(row0, RELU_BLK)],
                            rows_v.at[buf, pl.ds(0, RELU_BLK)])

            @plsc.parallel_loop(0, RELU_BLK, unroll=4)
            def _(r):
                for k in range(D_HALF // 16):
                    sl = pl.ds(16 * k, 16)
                    rows_v[buf, r, sl] = jnp.maximum(rows_v[buf, r, sl], 0.0)

            pltpu.sync_copy(rows_v.at[buf, pl.ds(0, RELU_BLK)],
                            out_hbm.at[pl.ds(row0, RELU_BLK),
                                       pl.ds(c * D_HALF, D_HALF)])

    return spmm


def kernel(x, edge_index, adj_values, W):
    e = edge_index.shape[1]
    n_chunks = e // (NSUB * CHUNK)
    assert n_chunks * NSUB * CHUNK == e and n_chunks % NMETA == 0

    src = edge_index[0].reshape(NSUB, n_chunks, CHUNK)
    dst = edge_index[1].reshape(NSUB, n_chunks, CHUNK)
    val = adj_values.reshape(NSUB, n_chunks, CHUNK)

    w_split = W.reshape(D_IN, NSC, D_HALF).transpose(1, 0, 2)
    h_split = _matmul_split(x, w_split, row_blk=1000)
    h_flat = h_split.reshape(NSC * N, D_HALF)

    return _make_sc_kernel(n_chunks)(h_flat, src, dst, val)


# speedup vs baseline: 1.4209x; 1.1423x over previous
import functools

import jax
import jax.numpy as jnp
from jax import lax
from jax.experimental import pallas as pl
from jax.experimental.pallas import tpu as pltpu
from jax.experimental.pallas import tpu_sc as plsc

N = 10000
D_IN = 128
D_OUT = 128
D_HALF = D_OUT // 2
NSC = 2
NSUB = 16
CHUNK = 128
ROWS_PER_SUB = N // NSUB
RELU_BLK = 125
NBUF = 3
NMETA = 6


def _matmul_body(x_ref, w_ref, o_ref):
    o_ref[0] = jnp.dot(x_ref[...], w_ref[0], preferred_element_type=jnp.float32)


def _matmul_split(x, w_split, row_blk):
    n = x.shape[0]
    grid = (NSC, n // row_blk)
    return pl.pallas_call(
        _matmul_body,
        grid=grid,
        in_specs=[
            pl.BlockSpec((row_blk, D_IN), lambda c, i: (i, 0)),
            pl.BlockSpec((1, D_IN, D_HALF), lambda c, i: (c, 0, 0)),
        ],
        out_specs=pl.BlockSpec((1, row_blk, D_HALF), lambda c, i: (c, i, 0)),
        out_shape=jax.ShapeDtypeStruct((NSC, n, D_HALF), jnp.float32),
    )(x, w_split)


def _make_sc_kernel(n_chunks):
    assert n_chunks % NMETA == 0
    mesh = plsc.VectorSubcoreMesh(core_axis_name="c", subcore_axis_name="s")

    @functools.partial(
        pl.kernel,
        mesh=mesh,
        out_type=jax.ShapeDtypeStruct((N, D_OUT), jnp.float32),
        compiler_params=pltpu.CompilerParams(
            use_tc_tiling_on_sc=False, needs_layout_passes=False),
        scratch_types=[
            pltpu.VMEM((NMETA, 3, CHUNK), jnp.int32),
            pltpu.VMEM((NBUF, CHUNK, D_HALF), jnp.float32),
            pltpu.VMEM_SHARED((N, D_HALF), jnp.float32),
            pltpu.VMEM_SHARED((N, D_HALF), jnp.float32),
            pltpu.SemaphoreType.DMA((NMETA,)),
            pltpu.SemaphoreType.DMA((NBUF,)),
            pltpu.SemaphoreType.DMA((NBUF,)),
        ],
    )
    def spmm(h_hbm, meta_hbm, out_hbm,
             meta_v, rows_v, h_spm, acc, msem, gsem, ssem):
        c = lax.axis_index("c")
        s = lax.axis_index("s")

        h_base = c * N + s * ROWS_PER_SUB
        pltpu.sync_copy(h_hbm.at[pl.ds(h_base, ROWS_PER_SUB)],
                        h_spm.at[pl.ds(s * ROWS_PER_SUB, ROWS_PER_SUB)])

        @plsc.parallel_loop(0, CHUNK, unroll=4)
        def _(i):
            for k in range(D_HALF // 16):
                rows_v[0, i, pl.ds(16 * k, 16)] = jnp.zeros((16,), jnp.float32)

        for b in range(ROWS_PER_SUB // RELU_BLK):
            pltpu.sync_copy(
                rows_v.at[0, pl.ds(0, RELU_BLK)],
                acc.at[pl.ds(s * ROWS_PER_SUB + b * RELU_BLK, RELU_BLK)],
            )
        plsc.subcore_barrier()

        def start_meta(j, mj):
            pltpu.async_copy(meta_hbm.at[s, j], meta_v.at[mj], msem.at[mj])

        def start_gather(b, mj):
            pltpu.async_copy(h_spm.at[meta_v.at[mj, 0]], rows_v.at[b],
                             gsem.at[b])

        for mj in range(NMETA):
            start_meta(mj, mj)
        for b in range(NBUF):
            pltpu.make_async_copy(meta_hbm.at[s, b], meta_v.at[b],
                                  msem.at[b]).wait()
            start_gather(b, b)

        def process(j, b, mj):
            pltpu.make_async_copy(h_spm.at[meta_v.at[mj, 0]], rows_v.at[b],
                                  gsem.at[b]).wait()

            @plsc.parallel_loop(0, CHUNK // 16, unroll=2)
            def _(m):
                v16 = plsc.bitcast(meta_v[mj, 2, pl.ds(m * 16, 16)], jnp.float32)
                for r2 in range(16):
                    bc = jnp.broadcast_to(v16[r2], (16,))
                    r = m * 16 + r2
                    for k in range(D_HALF // 16):
                        sl = pl.ds(16 * k, 16)
                        rows_v[b, r, sl] = rows_v[b, r, sl] * bc

            pltpu.async_copy(rows_v.at[b], acc.at[meta_v.at[mj, 1]],
                             ssem.at[b], add=True)

        def ring_body(g, _):
            for u in range(NMETA):
                j = g * NMETA + u
                b = u % NBUF
                process(j, b, u)
                br = (b + 2) % NBUF
                mr = (u + 5) % NMETA
                mg = (u + 2) % NMETA

                @pl.when(jnp.logical_and(j >= 1, j + 2 < n_chunks))
                def _():
                    pltpu.make_async_copy(rows_v.at[br], acc.at[meta_v.at[mr, 1]],
                                          ssem.at[br]).wait()

                    @pl.when(j + 5 < n_chunks)
                    def _():
                        start_meta(j + 5, mr)

                    pltpu.make_async_copy(meta_hbm.at[s, j + 2],
                                          meta_v.at[mg], msem.at[mg]).wait()
                    start_gather(br, mg)
            return ()

        lax.fori_loop(0, n_chunks // NMETA, ring_body, ())

        for b in range(NBUF):
            mj = (n_chunks - NBUF + b) % NMETA
            pltpu.make_async_copy(rows_v.at[b], acc.at[meta_v.at[mj, 1]],
                                  ssem.at[b]).wait()
        plsc.subcore_barrier()

        for b in range(ROWS_PER_SUB // RELU_BLK):
            row0 = s * ROWS_PER_SUB + b * RELU_BLK
            buf = b % NBUF
            pltpu.sync_copy(acc.at[pl.ds(row0, RELU_BLK)],
                            rows_v.at[buf, pl.ds(0, RELU_BLK)])

            @plsc.parallel_loop(0, RELU_BLK, unroll=4)
            def _(r):
                for k in range(D_HALF // 16):
                    sl = pl.ds(16 * k, 16)
                    rows_v[buf, r, sl] = jnp.maximum(rows_v[buf, r, sl], 0.0)

            pltpu.sync_copy(rows_v.at[buf, pl.ds(0, RELU_BLK)],
                            out_hbm.at[pl.ds(row0, RELU_BLK),
                                       pl.ds(c * D_HALF, D_HALF)])

    return spmm


def kernel(x, edge_index, adj_values, W):
    e = edge_index.shape[1]
    n_chunks = -(-e // (NSUB * CHUNK))
    n_chunks = -(-n_chunks // NMETA) * NMETA
    e_pad = NSUB * n_chunks * CHUNK
    pad = e_pad - e

    src = jnp.concatenate([edge_index[0], jnp.zeros((pad,), jnp.int32)])
    dst = jnp.concatenate([edge_index[1], jnp.zeros((pad,), jnp.int32)])
    val = jnp.concatenate([adj_values, jnp.zeros((pad,), jnp.float32)])
    vali = lax.bitcast_convert_type(val, jnp.int32)
    meta = jnp.stack(
        [src.reshape(NSUB, n_chunks, CHUNK),
         dst.reshape(NSUB, n_chunks, CHUNK),
         vali.reshape(NSUB, n_chunks, CHUNK)], axis=2)

    w_split = W.reshape(D_IN, NSC, D_HALF).transpose(1, 0, 2)
    h_split = _matmul_split(x, w_split, row_blk=1000)
    h_flat = h_split.reshape(NSC * N, D_HALF)

    return _make_sc_kernel(n_chunks)(h_flat, meta)


# speedup vs baseline: 1.4660x; 1.0318x over previous
import functools

import jax
import jax.numpy as jnp
from jax import lax
from jax.experimental import pallas as pl
from jax.experimental.pallas import tpu as pltpu
from jax.experimental.pallas import tpu_sc as plsc

N = 10000
D_IN = 128
D_OUT = 128
D_HALF = D_OUT // 2
NSC = 2
NSUB = 16
CHUNK = 128
ROWS_PER_SUB = N // NSUB
RELU_BLK = 125
NBUF = 4
NMETA = 8


def _matmul_body(x_ref, w_ref, o_ref):
    o_ref[0] = jnp.dot(x_ref[...], w_ref[0], preferred_element_type=jnp.float32)


def _matmul_split(x, w_split, row_blk):
    n = x.shape[0]
    grid = (NSC, n // row_blk)
    return pl.pallas_call(
        _matmul_body,
        grid=grid,
        in_specs=[
            pl.BlockSpec((row_blk, D_IN), lambda c, i: (i, 0)),
            pl.BlockSpec((1, D_IN, D_HALF), lambda c, i: (c, 0, 0)),
        ],
        out_specs=pl.BlockSpec((1, row_blk, D_HALF), lambda c, i: (c, i, 0)),
        out_shape=jax.ShapeDtypeStruct((NSC, n, D_HALF), jnp.float32),
    )(x, w_split)


def _make_sc_kernel(n_chunks):
    assert n_chunks % NMETA == 0
    mesh = plsc.VectorSubcoreMesh(core_axis_name="c", subcore_axis_name="s")

    @functools.partial(
        pl.kernel,
        mesh=mesh,
        out_type=jax.ShapeDtypeStruct((N, D_OUT), jnp.float32),
        compiler_params=pltpu.CompilerParams(
            use_tc_tiling_on_sc=False, needs_layout_passes=False),
        scratch_types=[
            pltpu.VMEM((NMETA, 3, CHUNK), jnp.int32),
            pltpu.VMEM((NBUF, CHUNK, D_HALF), jnp.float32),
            pltpu.VMEM_SHARED((N, D_HALF), jnp.float32),
            pltpu.VMEM_SHARED((N, D_HALF), jnp.float32),
            pltpu.SemaphoreType.DMA((NMETA,)),
            pltpu.SemaphoreType.DMA((NBUF,)),
            pltpu.SemaphoreType.DMA((NBUF,)),
        ],
    )
    def spmm(h_hbm, meta_hbm, out_hbm,
             meta_v, rows_v, h_spm, acc, msem, gsem, ssem):
        c = lax.axis_index("c")
        s = lax.axis_index("s")

        h_base = c * N + s * ROWS_PER_SUB
        pltpu.sync_copy(h_hbm.at[pl.ds(h_base, ROWS_PER_SUB)],
                        h_spm.at[pl.ds(s * ROWS_PER_SUB, ROWS_PER_SUB)])

        @plsc.parallel_loop(0, CHUNK, unroll=4)
        def _(i):
            for k in range(D_HALF // 16):
                rows_v[0, i, pl.ds(16 * k, 16)] = jnp.zeros((16,), jnp.float32)

        for b in range(ROWS_PER_SUB // RELU_BLK):
            pltpu.sync_copy(
                rows_v.at[0, pl.ds(0, RELU_BLK)],
                acc.at[pl.ds(s * ROWS_PER_SUB + b * RELU_BLK, RELU_BLK)],
            )
        plsc.subcore_barrier()

        def start_meta(j, mj):
            pltpu.async_copy(meta_hbm.at[s, j], meta_v.at[mj], msem.at[mj])

        def start_gather(b, mj):
            pltpu.async_copy(h_spm.at[meta_v.at[mj, 0]], rows_v.at[b],
                             gsem.at[b])

        for mj in range(NMETA):
            start_meta(mj, mj)
        for b in range(NBUF):
            pltpu.make_async_copy(meta_hbm.at[s, b], meta_v.at[b],
                                  msem.at[b]).wait()
            start_gather(b, b)

        def process(j, b, mj):
            pltpu.make_async_copy(h_spm.at[meta_v.at[mj, 0]], rows_v.at[b],
                                  gsem.at[b]).wait()

            @plsc.parallel_loop(0, CHUNK // 16, unroll=2)
            def _(m):
                v16 = plsc.bitcast(meta_v[mj, 2, pl.ds(m * 16, 16)], jnp.float32)
                for r2 in range(16):
                    bc = jnp.broadcast_to(v16[r2], (16,))
                    r = m * 16 + r2
                    for k in range(D_HALF // 16):
                        sl = pl.ds(16 * k, 16)
                        rows_v[b, r, sl] = rows_v[b, r, sl] * bc

            pltpu.async_copy(rows_v.at[b], acc.at[meta_v.at[mj, 1]],
                             ssem.at[b], add=True)

        def ring_body(g, _):
            for u in range(NMETA):
                j = g * NMETA + u
                b = u % NBUF
                process(j, b, u)
                br = (b + NBUF - 1) % NBUF
                mr = (u + NMETA - 1) % NMETA
                mg = (u + NBUF - 1) % NMETA

                @pl.when(jnp.logical_and(j >= 1, j + NBUF - 1 < n_chunks))
                def _():
                    pltpu.make_async_copy(rows_v.at[br], acc.at[meta_v.at[mr, 1]],
                                          ssem.at[br]).wait()

                    @pl.when(j + NMETA - 1 < n_chunks)
                    def _():
                        start_meta(j + NMETA - 1, mr)

                    pltpu.make_async_copy(meta_hbm.at[s, j + NBUF - 1],
                                          meta_v.at[mg], msem.at[mg]).wait()
                    start_gather(br, mg)
            return ()

        lax.fori_loop(0, n_chunks // NMETA, ring_body, ())

        for b in range(NBUF):
            mj = (n_chunks - NBUF + b) % NMETA
            pltpu.make_async_copy(rows_v.at[b], acc.at[meta_v.at[mj, 1]],
                                  ssem.at[b]).wait()
        plsc.subcore_barrier()

        for b in range(ROWS_PER_SUB // RELU_BLK):
            row0 = s * ROWS_PER_SUB + b * RELU_BLK
            buf = b % NBUF
            pltpu.sync_copy(acc.at[pl.ds(row0, RELU_BLK)],
                            rows_v.at[buf, pl.ds(0, RELU_BLK)])

            @plsc.parallel_loop(0, RELU_BLK, unroll=4)
            def _(r):
                for k in range(D_HALF // 16):
                    sl = pl.ds(16 * k, 16)
                    rows_v[buf, r, sl] = jnp.maximum(rows_v[buf, r, sl], 0.0)

            pltpu.sync_copy(rows_v.at[buf, pl.ds(0, RELU_BLK)],
                            out_hbm.at[pl.ds(row0, RELU_BLK),
                                       pl.ds(c * D_HALF, D_HALF)])

    return spmm


def kernel(x, edge_index, adj_values, W):
    e = edge_index.shape[1]
    n_chunks = -(-e // (NSUB * CHUNK))
    n_chunks = -(-n_chunks // NMETA) * NMETA
    e_pad = NSUB * n_chunks * CHUNK
    pad = e_pad - e

    src = jnp.concatenate([edge_index[0], jnp.zeros((pad,), jnp.int32)])
    dst = jnp.concatenate([edge_index[1], jnp.zeros((pad,), jnp.int32)])
    val = jnp.concatenate([adj_values, jnp.zeros((pad,), jnp.float32)])
    vali = lax.bitcast_convert_type(val, jnp.int32)
    meta = jnp.stack(
        [src.reshape(NSUB, n_chunks, CHUNK),
         dst.reshape(NSUB, n_chunks, CHUNK),
         vali.reshape(NSUB, n_chunks, CHUNK)], axis=2)

    w_split = W.reshape(D_IN, NSC, D_HALF).transpose(1, 0, 2)
    h_split = _matmul_split(x, w_split, row_blk=1000)
    h_flat = h_split.reshape(NSC * N, D_HALF)

    return _make_sc_kernel(n_chunks)(h_flat, meta)
